# baseline (speedup 1.0000x reference)
import functools

import jax
import jax.numpy as jnp
from jax import lax
from jax.experimental import pallas as pl
from jax.experimental.pallas import tpu as pltpu
from jax.experimental.pallas import tpu_sc as plsc

N = 100000
D = 128
K = 5
B = 16384
NW = 32
PER_W = B // NW
CH = 128
NCH = PER_W // CH
NB = 3


def _sc_gather_one(state, last_t, idxp, idx):
    mesh = plsc.VectorSubcoreMesh(core_axis_name="c", subcore_axis_name="s")

    @functools.partial(
        pl.kernel,
        out_type=(
            jax.ShapeDtypeStruct((B // 2, D), jnp.uint32),
            jax.ShapeDtypeStruct((B,), jnp.float32),
        ),
        mesh=mesh,
        compiler_params=pltpu.CompilerParams(use_tc_tiling_on_sc=False,
                                             needs_layout_passes=False),
        scratch_types=[
            pltpu.VMEM((PER_W,), jnp.int32),
            pltpu.VMEM((PER_W,), jnp.int32),
            pltpu.VMEM((NB, CH, D), jnp.float32),
            pltpu.VMEM((CH // 2, D), jnp.uint32),
            pltpu.VMEM((PER_W,), jnp.float32),
            pltpu.SemaphoreType.DMA,
            pltpu.SemaphoreType.DMA,
            pltpu.SemaphoreType.DMA,
            pltpu.SemaphoreType.DMA,
        ],
    )
    def k(state_hbm, lastt_hbm, idxp_hbm, idx_hbm, rows_out, lt_out, idxp_v,
          idx_v, rows_v, rows_w, lt_v, sem_lt, s0, s1, s2):
        sems = (s0, s1, s2)
        wid = lax.axis_index("s") * 2 + lax.axis_index("c")
        base_w = wid * PER_W
        pltpu.sync_copy(idxp_hbm.at[pl.ds(base_w, PER_W)], idxp_v)
        pltpu.sync_copy(idx_hbm.at[pl.ds(base_w, PER_W)], idx_v)
        lt_cps = [
            pltpu.async_copy(lastt_hbm.at[idx_v.at[pl.ds(j * CH, CH)]],
                             lt_v.at[pl.ds(j * CH, CH)], sem_lt)
            for j in range(NCH)
        ]
        gcp = [None] * NCH
        for j in range(NB - 1):
            gcp[j] = pltpu.async_copy(
                state_hbm.at[idxp_v.at[pl.ds(j * CH, CH)]],
                rows_v.at[j % NB], sems[j % NB])

        def _pack_chunk(b):
            def body(t, carry):
                rb = pl.multiple_of(8 * t, 8)
                for s in range(8):
                    for g in range(D // 16):
                        lo = plsc.bitcast(
                            rows_v[b, rb + s, pl.ds(g * 16, 16)], jnp.uint32)
                        hi = plsc.bitcast(
                            rows_v[b, rb + s + 64, pl.ds(g * 16, 16)],
                            jnp.uint32)
                        rows_w[rb + s, pl.ds(g * 16, 16)] = (
                            ((lo + 0x8000) >> 16)
                            | ((hi + 0x8000) & jnp.uint32(0xFFFF0000)))
                return carry
            lax.fori_loop(0, CH // 16, body, 0)

        for j in range(NCH):
            nxt = j + NB - 1
            if nxt < NCH:
                gcp[nxt] = pltpu.async_copy(
                    state_hbm.at[idxp_v.at[pl.ds(nxt * CH, CH)]],
                    rows_v.at[nxt % NB], sems[nxt % NB])
            gcp[j].wait()
            _pack_chunk(j % NB)
            pltpu.sync_copy(
                rows_w,
                rows_out.at[pl.ds((base_w + j * CH) // 2, CH // 2)])
        for cp in lt_cps:
            cp.wait()
        pltpu.sync_copy(lt_v, lt_out.at[pl.ds(base_w, PER_W)])

    return k(state, last_t, idxp, idx)


_RB = 2048
_NBLK = B // _RB


def _tc_body(ld_ref, rows_ref, lt_ref, t_ref, w1t_ref, b1_ref, w2t_ref,
             b2_ref, out_ref):
    ltv = lt_ref[...]
    tv = t_ref[...]
    dt = jnp.maximum(tv - ltv, 0.0)
    ld = ld_ref[0, 0]
    decay = jnp.log1p(jnp.exp(jnp.full((128,), ld, jnp.float32)))[0]
    gate = jnp.exp(-decay * dt)
    x = pltpu.bitcast(rows_ref[...], jnp.bfloat16)
    yt = lax.dot_general(w1t_ref[...], x, (((1,), (1,)), ((), ())),
                         preferred_element_type=jnp.float32)
    ht = jnp.maximum(yt * gate[None, :] + b1_ref[...], 0.0)
    logits_t = jnp.dot(w2t_ref[...], ht.astype(jnp.bfloat16),
                       preferred_element_type=jnp.float32) + b2_ref[...]
    m = jnp.max(logits_t, axis=0, keepdims=True)
    e = jnp.exp(logits_t - m)
    out_ref[...] = e / jnp.sum(e, axis=0, keepdims=True)


def _tc_mlp_one(rows, lt_g, t, ld, w1t, b1c, w2t, b2c, interpret=False):
    return pl.pallas_call(
        _tc_body,
        grid=(_NBLK,),
        in_specs=[
            pl.BlockSpec(memory_space=pltpu.SMEM),
            pl.BlockSpec((_RB // 2, D), lambda i: (i, 0)),
            pl.BlockSpec((_RB,), lambda i: (i,)),
            pl.BlockSpec((_RB,), lambda i: (i,)),
            pl.BlockSpec((D, D), lambda i: (0, 0)),
            pl.BlockSpec((D, 1), lambda i: (0, 0)),
            pl.BlockSpec((K, D), lambda i: (0, 0)),
            pl.BlockSpec((K, 1), lambda i: (0, 0)),
        ],
        out_specs=pl.BlockSpec((K, _RB), lambda i: (0, i)),
        out_shape=jax.ShapeDtypeStruct((K, B), jnp.float32),
        interpret=interpret,
    )(ld, rows, lt_g, t, w1t, b1c, w2t, b2c)


def kernel(source_nodes, destination_nodes, negative_nodes, edge_times,
           edge_idxs, state, last_t, log_decay, W1, b1, W2, b2):
    ld = jnp.reshape(log_decay, (1, 1))
    w1t = W1.T.astype(jnp.bfloat16)
    w2t = W2.T.astype(jnp.bfloat16)
    b1c = b1.reshape(D, 1)
    b2c = b2.reshape(K, 1)
    outs = []
    for idx in (source_nodes, destination_nodes, negative_nodes):
        idxp = idx.reshape(B // CH, CH // 2, 2).transpose(0, 2, 1).reshape(B)
        rows, lt_g = _sc_gather_one(state, last_t, idxp, idx)
        outs.append(_tc_mlp_one(rows, lt_g, edge_times, ld,
                                w1t, b1c, w2t, b2c).T)
    return tuple(outs)


# speedup vs baseline: 1.0704x; 1.0704x over previous
import functools

import jax
import jax.numpy as jnp
from jax import lax
from jax.experimental import pallas as pl
from jax.experimental.pallas import tpu as pltpu
from jax.experimental.pallas import tpu_sc as plsc

N = 100000
D = 128
K = 5
B = 16384
NW = 32
PER_W = B // NW
CH = 128
NCH = PER_W // CH
NB = 3


def _sc_gather_one(state, last_t, idx):
    mesh = plsc.VectorSubcoreMesh(core_axis_name="c", subcore_axis_name="s")

    @functools.partial(
        pl.kernel,
        out_type=(
            jax.ShapeDtypeStruct((B // 2, D), jnp.uint32),
            jax.ShapeDtypeStruct((B,), jnp.float32),
        ),
        mesh=mesh,
        compiler_params=pltpu.CompilerParams(use_tc_tiling_on_sc=False,
                                             needs_layout_passes=False),
        scratch_types=[
            pltpu.VMEM((PER_W,), jnp.int32),
            pltpu.VMEM((PER_W,), jnp.int32),
            pltpu.VMEM((NB, CH, D), jnp.float32),
            pltpu.VMEM((CH // 2, D), jnp.uint32),
            pltpu.VMEM((PER_W,), jnp.float32),
            pltpu.SemaphoreType.DMA,
            pltpu.SemaphoreType.DMA,
            pltpu.SemaphoreType.DMA,
            pltpu.SemaphoreType.DMA,
        ],
    )
    def k(state_hbm, lastt_hbm, idx_hbm, rows_out, lt_out, idxp_v,
          idx_v, rows_v, rows_w, lt_v, sem_lt, s0, s1, s2):
        sems = (s0, s1, s2)
        wid = lax.axis_index("s") * 2 + lax.axis_index("c")
        base_w = wid * PER_W
        pltpu.sync_copy(idx_hbm.at[pl.ds(base_w, PER_W)], idx_v)
        iota16 = jnp.arange(16, dtype=jnp.int32)
        for j in range(NCH):
            for h in range(2):
                for g in range(4):
                    src = iota16 * 2 + (j * CH + g * 32 + h)
                    idxp_v[pl.ds(j * CH + h * 64 + g * 16, 16)] = (
                        plsc.load_gather(idx_v, [src]))
        lt_cps = [
            pltpu.async_copy(lastt_hbm.at[idx_v.at[pl.ds(j * CH, CH)]],
                             lt_v.at[pl.ds(j * CH, CH)], sem_lt)
            for j in range(NCH)
        ]
        gcp = [None] * NCH
        for j in range(NB - 1):
            gcp[j] = pltpu.async_copy(
                state_hbm.at[idxp_v.at[pl.ds(j * CH, CH)]],
                rows_v.at[j % NB], sems[j % NB])

        def _pack_chunk(b):
            def body(t, carry):
                rb = pl.multiple_of(8 * t, 8)
                for s in range(8):
                    for g in range(D // 16):
                        lo = plsc.bitcast(
                            rows_v[b, rb + s, pl.ds(g * 16, 16)], jnp.uint32)
                        hi = plsc.bitcast(
                            rows_v[b, rb + s + 64, pl.ds(g * 16, 16)],
                            jnp.uint32)
                        rows_w[rb + s, pl.ds(g * 16, 16)] = (
                            ((lo + 0x8000) >> 16)
                            | ((hi + 0x8000) & jnp.uint32(0xFFFF0000)))
                return carry
            lax.fori_loop(0, CH // 16, body, 0)

        for j in range(NCH):
            nxt = j + NB - 1
            if nxt < NCH:
                gcp[nxt] = pltpu.async_copy(
                    state_hbm.at[idxp_v.at[pl.ds(nxt * CH, CH)]],
                    rows_v.at[nxt % NB], sems[nxt % NB])
            gcp[j].wait()
            _pack_chunk(j % NB)
            pltpu.sync_copy(
                rows_w,
                rows_out.at[pl.ds((base_w + j * CH) // 2, CH // 2)])
        for cp in lt_cps:
            cp.wait()
        pltpu.sync_copy(lt_v, lt_out.at[pl.ds(base_w, PER_W)])

    return k(state, last_t, idx)


_RB = 2048
_NBLK = B // _RB


def _tc_body(ld_ref, rows_ref, lt_ref, t_ref, w1t_ref, b1_ref, w2t_ref,
             b2_ref, out_ref):
    ltv = lt_ref[...]
    tv = t_ref[...]
    dt = jnp.maximum(tv - ltv, 0.0)
    ld = ld_ref[0, 0]
    decay = jnp.log1p(jnp.exp(jnp.full((128,), ld, jnp.float32)))[0]
    gate = jnp.exp(-decay * dt)
    x = pltpu.bitcast(rows_ref[...], jnp.bfloat16)
    yt = lax.dot_general(w1t_ref[...], x, (((1,), (1,)), ((), ())),
                         preferred_element_type=jnp.float32)
    ht = jnp.maximum(yt * gate[None, :] + b1_ref[...], 0.0)
    logits_t = jnp.dot(w2t_ref[...], ht.astype(jnp.bfloat16),
                       preferred_element_type=jnp.float32) + b2_ref[...]
    m = jnp.max(logits_t, axis=0, keepdims=True)
    e = jnp.exp(logits_t - m)
    out_ref[...] = e / jnp.sum(e, axis=0, keepdims=True)


def _tc_mlp_one(rows, lt_g, t, ld, w1t, b1c, w2t, b2c, interpret=False):
    return pl.pallas_call(
        _tc_body,
        grid=(_NBLK,),
        in_specs=[
            pl.BlockSpec(memory_space=pltpu.SMEM),
            pl.BlockSpec((_RB // 2, D), lambda i: (i, 0)),
            pl.BlockSpec((_RB,), lambda i: (i,)),
            pl.BlockSpec((_RB,), lambda i: (i,)),
            pl.BlockSpec((D, D), lambda i: (0, 0)),
            pl.BlockSpec((D, 1), lambda i: (0, 0)),
            pl.BlockSpec((K, D), lambda i: (0, 0)),
            pl.BlockSpec((K, 1), lambda i: (0, 0)),
        ],
        out_specs=pl.BlockSpec((K, _RB), lambda i: (0, i)),
        out_shape=jax.ShapeDtypeStruct((K, B), jnp.float32),
        interpret=interpret,
    )(ld, rows, lt_g, t, w1t, b1c, w2t, b2c)


def kernel(source_nodes, destination_nodes, negative_nodes, edge_times,
           edge_idxs, state, last_t, log_decay, W1, b1, W2, b2):
    ld = jnp.reshape(log_decay, (1, 1))
    w1t = W1.T.astype(jnp.bfloat16)
    w2t = W2.T.astype(jnp.bfloat16)
    b1c = b1.reshape(D, 1)
    b2c = b2.reshape(K, 1)
    outs = []
    for idx in (source_nodes, destination_nodes, negative_nodes):
        rows, lt_g = _sc_gather_one(state, last_t, idx)
        outs.append(_tc_mlp_one(rows, lt_g, edge_times, ld,
                                w1t, b1c, w2t, b2c).T)
    return tuple(outs)


# speedup vs baseline: 1.1904x; 1.1120x over previous
import functools

import jax
import jax.numpy as jnp
from jax import lax
from jax.experimental import pallas as pl
from jax.experimental.pallas import tpu as pltpu
from jax.experimental.pallas import tpu_sc as plsc

N = 100000
D = 128
K = 5
B = 16384
NW = 32
PER_W = B // NW
CH = 128
NCH = PER_W // CH
NB = 3


def _sc_gather_one(state, last_t, idx):
    mesh = plsc.VectorSubcoreMesh(core_axis_name="c", subcore_axis_name="s")

    @functools.partial(
        pl.kernel,
        out_type=(
            jax.ShapeDtypeStruct((B, D), jnp.float32),
            jax.ShapeDtypeStruct((B,), jnp.float32),
        ),
        mesh=mesh,
        scratch_types=[
            pltpu.VMEM((PER_W,), jnp.int32),
            pltpu.VMEM((NB, CH, D), jnp.float32),
            pltpu.VMEM((PER_W,), jnp.float32),
            pltpu.SemaphoreType.DMA,
            pltpu.SemaphoreType.DMA,
            pltpu.SemaphoreType.DMA,
            pltpu.SemaphoreType.DMA,
            pltpu.SemaphoreType.DMA,
            pltpu.SemaphoreType.DMA,
        ],
    )
    def k(state_hbm, lastt_hbm, idx_hbm, rows_out, lt_out, idx_v, rows_v,
          lt_v, sem_lt, s0, s1, s2, w0, w1):
        sems = (s0, s1, s2)
        wsems = (w0, w1)
        wid = lax.axis_index("s") * 2 + lax.axis_index("c")
        base_w = wid * PER_W
        pltpu.sync_copy(idx_hbm.at[pl.ds(base_w, PER_W)], idx_v)
        lt_cps = [
            pltpu.async_copy(lastt_hbm.at[idx_v.at[pl.ds(j * CH, CH)]],
                             lt_v.at[pl.ds(j * CH, CH)], sem_lt)
            for j in range(NCH)
        ]
        gcp = [None] * NCH
        wcp = [None] * NCH
        for j in range(NB - 1):
            gcp[j] = pltpu.async_copy(
                state_hbm.at[idx_v.at[pl.ds(j * CH, CH)]],
                rows_v.at[j % NB], sems[j % NB])
        for j in range(NCH):
            nxt = j + NB - 1
            if nxt < NCH:
                if nxt - NB >= 0:
                    wcp[nxt - NB].wait()
                gcp[nxt] = pltpu.async_copy(
                    state_hbm.at[idx_v.at[pl.ds(nxt * CH, CH)]],
                    rows_v.at[nxt % NB], sems[nxt % NB])
            gcp[j].wait()
            wcp[j] = pltpu.async_copy(
                rows_v.at[j % NB],
                rows_out.at[pl.ds(base_w + j * CH, CH)], wsems[j % 2])
        for j in range(max(0, NCH - NB), NCH):
            wcp[j].wait()
        for cp in lt_cps:
            cp.wait()
        pltpu.sync_copy(lt_v, lt_out.at[pl.ds(base_w, PER_W)])

    return k(state, last_t, idx)


_RB = 2048
_NBLK = B // _RB


def _tc_body(ld_ref, rows_ref, lt_ref, t_ref, w1t_ref, b1_ref, w2t_ref,
             b2_ref, out_ref):
    ltv = lt_ref[...]
    tv = t_ref[...]
    dt = jnp.maximum(tv - ltv, 0.0)
    ld = ld_ref[0, 0]
    decay = jnp.log1p(jnp.exp(jnp.full((128,), ld, jnp.float32)))[0]
    gate = jnp.exp(-decay * dt)
    x = rows_ref[...].astype(jnp.bfloat16)
    yt = lax.dot_general(w1t_ref[...], x, (((1,), (1,)), ((), ())),
                         preferred_element_type=jnp.float32)
    ht = jnp.maximum(yt * gate[None, :] + b1_ref[...], 0.0)
    logits_t = jnp.dot(w2t_ref[...], ht.astype(jnp.bfloat16),
                       preferred_element_type=jnp.float32) + b2_ref[...]
    m = jnp.max(logits_t, axis=0, keepdims=True)
    e = jnp.exp(logits_t - m)
    out_ref[...] = e / jnp.sum(e, axis=0, keepdims=True)


def _tc_mlp_one(rows, lt_g, t, ld, w1t, b1c, w2t, b2c, interpret=False):
    return pl.pallas_call(
        _tc_body,
        grid=(_NBLK,),
        in_specs=[
            pl.BlockSpec(memory_space=pltpu.SMEM),
            pl.BlockSpec((_RB, D), lambda i: (i, 0)),
            pl.BlockSpec((_RB,), lambda i: (i,)),
            pl.BlockSpec((_RB,), lambda i: (i,)),
            pl.BlockSpec((D, D), lambda i: (0, 0)),
            pl.BlockSpec((D, 1), lambda i: (0, 0)),
            pl.BlockSpec((K, D), lambda i: (0, 0)),
            pl.BlockSpec((K, 1), lambda i: (0, 0)),
        ],
        out_specs=pl.BlockSpec((K, _RB), lambda i: (0, i)),
        out_shape=jax.ShapeDtypeStruct((K, B), jnp.float32),
        interpret=interpret,
    )(ld, rows, lt_g, t, w1t, b1c, w2t, b2c)


def kernel(source_nodes, destination_nodes, negative_nodes, edge_times,
           edge_idxs, state, last_t, log_decay, W1, b1, W2, b2):
    ld = jnp.reshape(log_decay, (1, 1))
    w1t = W1.T.astype(jnp.bfloat16)
    w2t = W2.T.astype(jnp.bfloat16)
    b1c = b1.reshape(D, 1)
    b2c = b2.reshape(K, 1)
    outs = []
    for idx in (source_nodes, destination_nodes, negative_nodes):
        rows, lt_g = _sc_gather_one(state, last_t, idx)
        outs.append(_tc_mlp_one(rows, lt_g, edge_times, ld,
                                w1t, b1c, w2t, b2c).T)
    return tuple(outs)
